# Initial kernel scaffold; baseline (speedup 1.0000x reference)
import jax
import jax.numpy as jnp
from jax.experimental import pallas as pl


def kernel(A, W, att_src, att_dst, bias):
    raise NotImplementedError("write your pallas kernel here")


# speedup vs baseline: 217.5594x; 217.5594x over previous
import jax
import jax.numpy as jnp
from jax.experimental import pallas as pl

_H = 4
_D = 128
_SLOPE = 0.2


def _lrelu(x):
    return jnp.where(x >= 0, x, x * _SLOPE)


def _gat_step_kernel(a_ref, w_ref, asrc_ref, adst_ref, bias_ref, out_ref):
    at = a_ref[0]
    x = at.T
    h = jnp.dot(x, w_ref[...], preferred_element_type=jnp.float32)
    acc = None
    for hd in range(_H):
        h_h = h[:, hd * _D:(hd + 1) * _D]
        asrc = asrc_ref[hd:hd + 1, :]
        adst = adst_ref[hd:hd + 1, :]
        a_src = jnp.dot(h_h, asrc.T, preferred_element_type=jnp.float32)
        a_dst = jnp.dot(h_h, adst.T, preferred_element_type=jnp.float32)
        e = _lrelu(a_dst + a_src.T)
        m = jnp.max(e, axis=1, keepdims=True)
        p = jnp.exp(e - m)
        s = jnp.sum(p, axis=1, keepdims=True)
        alpha = p / (s + 1e-16)
        o = jnp.dot(alpha, h_h, preferred_element_type=jnp.float32)
        acc = o if acc is None else acc + o
    out = acc * (1.0 / _H) + bias_ref[0:1, :]
    y = _lrelu(out).T
    n = y.shape[0]
    r = jax.lax.broadcasted_iota(jnp.int32, (n, n), 0)
    c = jax.lax.broadcasted_iota(jnp.int32, (n, n), 1)
    out_ref[0] = y + jnp.where(r == c, 1.0, 0.0).astype(y.dtype)


def kernel(A, W, att_src, att_dst, bias):
    T, _, N = A.shape
    bias2 = bias.reshape(1, -1)
    return pl.pallas_call(
        _gat_step_kernel,
        grid=(T,),
        in_specs=[
            pl.BlockSpec((1, N, N), lambda t: (t, 0, 0)),
            pl.BlockSpec(W.shape, lambda t: (0, 0)),
            pl.BlockSpec(att_src.shape, lambda t: (0, 0)),
            pl.BlockSpec(att_dst.shape, lambda t: (0, 0)),
            pl.BlockSpec(bias2.shape, lambda t: (0, 0)),
        ],
        out_specs=pl.BlockSpec((1, N, N), lambda t: (t, 0, 0)),
        out_shape=jax.ShapeDtypeStruct(A.shape, A.dtype),
    )(A, W, att_src, att_dst, bias2)


# speedup vs baseline: 260.7518x; 1.1985x over previous
import jax
import jax.numpy as jnp
from jax.experimental import pallas as pl

_H = 4
_D = 128
_SLOPE = 0.2
_B = 4


def _lrelu(x):
    return jnp.where(x >= 0, x, x * _SLOPE)


def _gat_kernel(a_ref, wt_ref, asrc_ref, adst_ref, bias_ref, out_ref):
    n = a_ref.shape[-1]
    xt = jnp.concatenate([a_ref[b] for b in range(_B)], axis=1)
    ht = jnp.dot(wt_ref[...], xt, preferred_element_type=jnp.float32)
    eye = jnp.where(
        jax.lax.broadcasted_iota(jnp.int32, (n, n), 0)
        == jax.lax.broadcasted_iota(jnp.int32, (n, n), 1),
        1.0, 0.0)
    for b in range(_B):
        acc = None
        for hd in range(_H):
            ht_hb = ht[hd * _D:(hd + 1) * _D, b * n:(b + 1) * n]
            a_src = jnp.dot(asrc_ref[hd:hd + 1, :], ht_hb,
                            preferred_element_type=jnp.float32)
            a_dst = jnp.dot(adst_ref[hd:hd + 1, :], ht_hb,
                            preferred_element_type=jnp.float32)
            et = _lrelu(a_src.T + a_dst)
            m = jnp.max(et, axis=0, keepdims=True)
            p = jnp.exp(et - m)
            s = jnp.sum(p, axis=0, keepdims=True)
            alpha_t = p / (s + 1e-16)
            o = jnp.dot(ht_hb, alpha_t, preferred_element_type=jnp.float32)
            acc = o if acc is None else acc + o
        out_t = acc * (1.0 / _H) + bias_ref[...]
        out_ref[b] = _lrelu(out_t) + eye


def kernel(A, W, att_src, att_dst, bias):
    T, _, N = A.shape
    wt = W.T
    bias_col = bias.reshape(-1, 1)
    return pl.pallas_call(
        _gat_kernel,
        grid=(T // _B,),
        in_specs=[
            pl.BlockSpec((_B, N, N), lambda t: (t, 0, 0)),
            pl.BlockSpec(wt.shape, lambda t: (0, 0)),
            pl.BlockSpec(att_src.shape, lambda t: (0, 0)),
            pl.BlockSpec(att_dst.shape, lambda t: (0, 0)),
            pl.BlockSpec(bias_col.shape, lambda t: (0, 0)),
        ],
        out_specs=pl.BlockSpec((_B, N, N), lambda t: (t, 0, 0)),
        out_shape=jax.ShapeDtypeStruct(A.shape, A.dtype),
    )(A, wt, att_src, att_dst, bias_col)


# speedup vs baseline: 531.5018x; 2.0383x over previous
import jax
import jax.numpy as jnp
from jax.experimental import pallas as pl

_H = 4
_D = 128
_SLOPE = 0.2
_B = 8


def _lrelu(x):
    return jnp.where(x >= 0, x, x * _SLOPE)


def _gat_kernel(a_ref, wt_ref, asrc_ref, adst_ref, bias_ref, out_ref):
    n = a_ref.shape[-1]
    xt = jnp.concatenate([a_ref[b] for b in range(_B)], axis=1)
    vsrc = jnp.concatenate([
        jnp.dot(asrc_ref[h:h + 1, :], wt_ref[h * _D:(h + 1) * _D, :],
                preferred_element_type=jnp.float32)
        for h in range(_H)], axis=0)
    vdst = jnp.concatenate([
        jnp.dot(adst_ref[h:h + 1, :], wt_ref[h * _D:(h + 1) * _D, :],
                preferred_element_type=jnp.float32)
        for h in range(_H)], axis=0)
    a_src_all = jnp.dot(vsrc, xt, preferred_element_type=jnp.float32)
    a_dst_all = jnp.dot(vdst, xt, preferred_element_type=jnp.float32)
    ht = jnp.dot(wt_ref[...], xt, preferred_element_type=jnp.float32)
    eye = jnp.where(
        jax.lax.broadcasted_iota(jnp.int32, (n, n), 0)
        == jax.lax.broadcasted_iota(jnp.int32, (n, n), 1),
        1.0, 0.0)
    for b in range(_B):
        acc = None
        for hd in range(_H):
            a_src = a_src_all[hd:hd + 1, b * n:(b + 1) * n]
            a_dst = a_dst_all[hd:hd + 1, b * n:(b + 1) * n]
            et = _lrelu(a_src.T + a_dst)
            m = _lrelu(jnp.max(a_src, axis=1, keepdims=True) + a_dst)
            p = jnp.exp(et - m)
            s = jnp.sum(p, axis=0, keepdims=True)
            r = 1.0 / (s + 1e-16)
            o = jnp.dot(ht[hd * _D:(hd + 1) * _D, b * n:(b + 1) * n], p,
                        preferred_element_type=jnp.float32) * r
            acc = o if acc is None else acc + o
        out_ref[b] = _lrelu(acc * (1.0 / _H) + bias_ref[...]) + eye


def kernel(A, W, att_src, att_dst, bias):
    T, _, N = A.shape
    wt = W.T
    bias_col = bias.reshape(-1, 1)
    return pl.pallas_call(
        _gat_kernel,
        grid=(T // _B,),
        in_specs=[
            pl.BlockSpec((_B, N, N), lambda t: (t, 0, 0)),
            pl.BlockSpec(wt.shape, lambda t: (0, 0)),
            pl.BlockSpec(att_src.shape, lambda t: (0, 0)),
            pl.BlockSpec(att_dst.shape, lambda t: (0, 0)),
            pl.BlockSpec(bias_col.shape, lambda t: (0, 0)),
        ],
        out_specs=pl.BlockSpec((_B, N, N), lambda t: (t, 0, 0)),
        out_shape=jax.ShapeDtypeStruct(A.shape, A.dtype),
    )(A, wt, att_src, att_dst, bias_col)
